# baseline (speedup 1.0000x reference)
import functools

import jax
import jax.numpy as jnp
from jax import lax
from jax.experimental import pallas as pl
from jax.experimental.pallas import tpu as pltpu
from jax.experimental.pallas import tpu_sc as plsc

NC = 2
NS = 16
LANES = 16
CHUNK = 80


def _transform_kernel(x_ref, w_ref, keep_ref, y_ref, init_ref):
    r_plus_1 = w_ref.shape[0]
    x = x_ref[...]
    for r in range(r_plus_1 - 1):
        y_ref[r] = jnp.dot(x, w_ref[r], preferred_element_type=jnp.float32)
    self_m = jnp.dot(x, w_ref[r_plus_1 - 1], preferred_element_type=jnp.float32)
    init_ref[...] = self_m * keep_ref[...]


def _sum_kernel(a_ref, b_ref, o_ref):
    o_ref[...] = a_ref[0] + b_ref[0]


def _make_sc_kernel(n, d, n_chunks_per_worker):
    mesh = plsc.VectorSubcoreMesh(core_axis_name="c", subcore_axis_name="s")
    nc_w = n_chunks_per_worker
    seg = d // LANES
    grp = CHUNK // LANES
    rpt_a = (-(-n // NS) + 7) // 8 * 8
    rpt_b = n - (NS - 1) * rpt_a
    assert rpt_b > 0 and rpt_b % 8 == 0
    assert nc_w >= 4 and nc_w % 2 == 1

    @functools.partial(
        pl.kernel,
        out_type=jax.ShapeDtypeStruct((NC, n, d), jnp.float32),
        mesh=mesh,
        scratch_types=[
            pltpu.VMEM_SHARED((n, d), jnp.float32),
            pltpu.VMEM((2, CHUNK), jnp.int32),
            pltpu.VMEM((2, CHUNK), jnp.int32),
            pltpu.VMEM((2, CHUNK), jnp.int32),
            pltpu.VMEM((2, CHUNK), jnp.int32),
            pltpu.VMEM((2, CHUNK), jnp.int32),
            pltpu.VMEM((2, CHUNK), jnp.int32),
            pltpu.VMEM((2, CHUNK), jnp.int32),
            pltpu.VMEM((2, CHUNK), jnp.float32),
            pltpu.VMEM((2, CHUNK, d), jnp.float32),
            pltpu.VMEM((2, CHUNK, d), jnp.float32),
            pltpu.SemaphoreType.DMA((2,)),
            pltpu.SemaphoreType.DMA((2,)),
            pltpu.SemaphoreType.DMA((2,)),
            pltpu.SemaphoreType.DMA((2,)),
            pltpu.SemaphoreType.DMA((2,)),
            pltpu.SemaphoreType.DMA((2,)),
        ],
    )
    def sc_kernel(y_hbm, init_hbm, src_hbm, dst_hbm, rel_hbm, w_hbm, out_hbm,
                  acc, sv, dv, rv, ia, ib, da, db, wv, ra, rb,
                  ed_sem, w_sem, ga_sem, gb_sem, sa_sem, sb_sem):
        c = lax.axis_index("c")
        s = lax.axis_index("s")
        wid = c * NS + s

        r0 = pl.multiple_of(s * rpt_a, 8)

        @pl.when(c == 0)
        def _():
            @pl.when(s < NS - 1)
            def _():
                pltpu.sync_copy(init_hbm.at[pl.ds(r0, rpt_a)],
                                acc.at[pl.ds(r0, rpt_a)])

            @pl.when(s == NS - 1)
            def _():
                pltpu.sync_copy(init_hbm.at[pl.ds(r0, rpt_b)],
                                acc.at[pl.ds(r0, rpt_b)])

        @pl.when(c == 1)
        def _():
            def zbody(i, _):
                for j in range(seg):
                    ra[0, i, pl.ds(j * LANES, LANES)] = jnp.zeros(
                        (LANES,), jnp.float32)
                return 0

            lax.fori_loop(0, CHUNK, zbody, 0, unroll=False)

            def zfill(row_cnt):
                full = row_cnt // CHUNK
                tail = row_cnt - full * CHUNK
                for t in range(full):
                    off = pl.multiple_of(r0 + t * CHUNK, 8)
                    pltpu.sync_copy(ra.at[0], acc.at[pl.ds(off, CHUNK)])
                if tail:
                    off = pl.multiple_of(r0 + full * CHUNK, 8)
                    pltpu.sync_copy(ra.at[0, pl.ds(0, tail)],
                                    acc.at[pl.ds(off, tail)])

            @pl.when(s < NS - 1)
            def _():
                zfill(rpt_a)

            @pl.when(s == NS - 1)
            def _():
                zfill(rpt_b)

        plsc.subcore_barrier()

        cbase = wid * nc_w

        def fetch_ed(k, slot):
            pltpu.async_copy(src_hbm.at[cbase + k], sv.at[slot],
                             ed_sem.at[slot])
            pltpu.async_copy(dst_hbm.at[cbase + k], dv.at[slot],
                             ed_sem.at[slot])
            pltpu.async_copy(rel_hbm.at[cbase + k], rv.at[slot],
                             ed_sem.at[slot])

        def fetch_w(k, slot):
            pltpu.async_copy(w_hbm.at[cbase + k], wv.at[slot],
                             w_sem.at[slot])

        def wait_ed(k, slot):
            pltpu.make_async_copy(src_hbm.at[cbase + k], sv.at[slot],
                                  ed_sem.at[slot]).wait()
            pltpu.make_async_copy(dst_hbm.at[cbase + k], dv.at[slot],
                                  ed_sem.at[slot]).wait()
            pltpu.make_async_copy(rel_hbm.at[cbase + k], rv.at[slot],
                                  ed_sem.at[slot]).wait()
            pltpu.make_async_copy(w_hbm.at[cbase + k], wv.at[slot],
                                  w_sem.at[slot]).wait()

        def compute_idx(slot):
            for g in range(grp):
                sl = pl.ds(g * LANES, LANES)
                src16 = sv[slot, sl]
                dst16 = dv[slot, sl]
                rbase = rv[slot, sl] * n
                ia[slot, sl] = rbase + src16
                ib[slot, sl] = rbase + dst16
                da[slot, sl] = dst16
                db[slot, sl] = src16

        def issue_gathers(slot):
            pltpu.async_copy(y_hbm.at[ia.at[slot]], ra.at[slot],
                             ga_sem.at[slot])
            pltpu.async_copy(y_hbm.at[ib.at[slot]], rb.at[slot],
                             gb_sem.at[slot])

        def scale(rows, slot):
            def body(g, _):
                w16 = wv[slot, pl.ds(g * LANES, LANES)]
                for l in range(LANES):
                    i = g * LANES + l
                    wb = jnp.full((LANES,), w16[l], jnp.float32)
                    for j in range(seg):
                        sl = pl.ds(j * LANES, LANES)
                        rows[slot, i, sl] = rows[slot, i, sl] * wb
                return 0

            lax.fori_loop(0, grp, body, 0, unroll=False)

        def wait_scatters(slot):
            pltpu.make_async_copy(ra.at[slot], acc.at[da.at[slot]],
                                  sa_sem.at[slot]).wait()
            pltpu.make_async_copy(rb.at[slot], acc.at[db.at[slot]],
                                  sb_sem.at[slot]).wait()

        def step(k, b, prep, wt, prefetch):
            nb = 1 - b
            if prep:
                if wt:
                    wait_scatters(nb)
                wait_ed(k + 1, nb)
                compute_idx(nb)
                if prefetch:
                    fetch_ed(k + 2, b)
                issue_gathers(nb)
            pltpu.make_async_copy(y_hbm.at[ia.at[b]], ra.at[b],
                                  ga_sem.at[b]).wait()
            scale(ra, b)
            pltpu.async_copy(ra.at[b], acc.at[da.at[b]], sa_sem.at[b],
                             add=True)
            pltpu.make_async_copy(y_hbm.at[ib.at[b]], rb.at[b],
                                  gb_sem.at[b]).wait()
            scale(rb, b)
            pltpu.async_copy(rb.at[b], acc.at[db.at[b]], sb_sem.at[b],
                             add=True)
            if prefetch:
                fetch_w(k + 2, b)

        fetch_ed(0, 0)
        fetch_w(0, 0)
        fetch_ed(1, 1)
        fetch_w(1, 1)
        wait_ed(0, 0)
        compute_idx(0)
        issue_gathers(0)
        step(0, 0, prep=True, wt=False, prefetch=True)

        def pair_body(t, _):
            step(2 * t + 1, 1, prep=True, wt=True, prefetch=True)
            step(2 * t + 2, 0, prep=True, wt=True, prefetch=True)
            return 0

        lax.fori_loop(0, (nc_w - 3) // 2, pair_body, 0, unroll=False)

        step(nc_w - 2, 1, prep=True, wt=True, prefetch=False)
        step(nc_w - 1, 0, prep=False, wt=False, prefetch=False)
        wait_scatters(1)
        wait_scatters(0)

        plsc.subcore_barrier()

        @pl.when(s < NS - 1)
        def _():
            pltpu.sync_copy(acc.at[pl.ds(r0, rpt_a)],
                            out_hbm.at[c, pl.ds(r0, rpt_a)])

        @pl.when(s == NS - 1)
        def _():
            pltpu.sync_copy(acc.at[pl.ds(r0, rpt_b)],
                            out_hbm.at[c, pl.ds(r0, rpt_b)])

    return sc_kernel


def kernel(x, node_keep_mask, source, target, edge_type, edge_weights, blocks):
    n, d = x.shape
    r_plus_1, nb, bs, _ = blocks.shape
    r_cnt = r_plus_1 - 1
    e = source.shape[0]

    w_dense = jnp.zeros((r_plus_1, d, d), dtype=jnp.float32)
    for b in range(nb):
        sl = slice(b * bs, (b + 1) * bs)
        w_dense = w_dense.at[:, sl, sl].set(blocks[:, b])

    keepf = node_keep_mask.astype(jnp.float32).reshape(n, 1)

    step = NC * NS * CHUNK
    ep = ((e + step - 1) // step) * step
    if ep // (NC * NS * CHUNK) % 2 == 0:
        ep += step
    if ep != e:
        pad = ep - e
        fill = (jnp.arange(pad, dtype=jnp.int32) * 7) % n
        source = jnp.concatenate([source.astype(jnp.int32), fill])
        target = jnp.concatenate([target.astype(jnp.int32), fill])
        edge_type = jnp.concatenate(
            [edge_type.astype(jnp.int32), jnp.zeros((pad,), jnp.int32)])
        edge_weights = jnp.concatenate(
            [edge_weights, jnp.zeros((pad,), jnp.float32)])
    n_chunks_per_worker = ep // (NC * NS * CHUNK)

    src_hbm = source.astype(jnp.int32).reshape(ep // CHUNK, CHUNK)
    dst_hbm = target.astype(jnp.int32).reshape(ep // CHUNK, CHUNK)
    rel_hbm = edge_type.astype(jnp.int32).reshape(ep // CHUNK, CHUNK)
    w_hbm = edge_weights.reshape(ep // CHUNK, CHUNK)

    tn = 400
    nt = n // tn

    y, init = pl.pallas_call(
        _transform_kernel,
        grid=(nt,),
        in_specs=[
            pl.BlockSpec((tn, d), lambda i: (i, 0)),
            pl.BlockSpec((r_plus_1, d, d), lambda i: (0, 0, 0)),
            pl.BlockSpec((tn, 1), lambda i: (i, 0)),
        ],
        out_specs=[
            pl.BlockSpec((r_cnt, tn, d), lambda i: (0, i, 0)),
            pl.BlockSpec((tn, d), lambda i: (i, 0)),
        ],
        out_shape=[
            jax.ShapeDtypeStruct((r_cnt, n, d), jnp.float32),
            jax.ShapeDtypeStruct((n, d), jnp.float32),
        ],
    )(x, w_dense, keepf)

    y_flat = y.reshape(r_cnt * n, d)

    sc = _make_sc_kernel(n, d, n_chunks_per_worker)
    partials = sc(y_flat, init, src_hbm, dst_hbm, rel_hbm, w_hbm)

    out = pl.pallas_call(
        _sum_kernel,
        grid=(nt,),
        in_specs=[
            pl.BlockSpec((1, tn, d), lambda i: (0, i, 0)),
            pl.BlockSpec((1, tn, d), lambda i: (1, i, 0)),
        ],
        out_specs=pl.BlockSpec((tn, d), lambda i: (i, 0)),
        out_shape=jax.ShapeDtypeStruct((n, d), jnp.float32),
    )(partials, partials)

    return out


# speedup vs baseline: 1.0601x; 1.0601x over previous
import functools

import jax
import jax.numpy as jnp
from jax import lax
from jax.experimental import pallas as pl
from jax.experimental.pallas import tpu as pltpu
from jax.experimental.pallas import tpu_sc as plsc

NC = 2
NS = 16
LANES = 16
CHUNK = 80


def _transform_kernel(x_ref, w_ref, keep_ref, y_ref, init_ref):
    r_plus_1 = w_ref.shape[0]
    x = x_ref[...]
    for r in range(r_plus_1 - 1):
        y_ref[r] = jnp.dot(x, w_ref[r], preferred_element_type=jnp.float32)
    self_m = jnp.dot(x, w_ref[r_plus_1 - 1], preferred_element_type=jnp.float32)
    init_ref[...] = self_m * keep_ref[...]


def _sum_kernel(a_ref, b_ref, o_ref):
    o_ref[...] = a_ref[0] + b_ref[0]


def _make_sc_kernel(n, d, n_chunks_per_worker):
    mesh = plsc.VectorSubcoreMesh(core_axis_name="c", subcore_axis_name="s")
    nc_w = n_chunks_per_worker
    seg = d // LANES
    grp = CHUNK // LANES
    rpt_a = (-(-n // NS) + 7) // 8 * 8
    rpt_b = n - (NS - 1) * rpt_a
    assert rpt_b > 0 and rpt_b % 8 == 0
    assert nc_w >= 4 and nc_w % 2 == 1

    @functools.partial(
        pl.kernel,
        out_type=jax.ShapeDtypeStruct((NC, n, d), jnp.float32),
        mesh=mesh,
        scratch_types=[
            pltpu.VMEM_SHARED((n, d), jnp.float32),
            pltpu.VMEM((2, CHUNK), jnp.int32),
            pltpu.VMEM((2, CHUNK), jnp.int32),
            pltpu.VMEM((2, CHUNK), jnp.int32),
            pltpu.VMEM((2, CHUNK), jnp.int32),
            pltpu.VMEM((2, CHUNK), jnp.int32),
            pltpu.VMEM((2, CHUNK), jnp.int32),
            pltpu.VMEM((2, CHUNK), jnp.int32),
            pltpu.VMEM((2, CHUNK), jnp.float32),
            pltpu.VMEM((2, CHUNK, d), jnp.float32),
            pltpu.VMEM((2, CHUNK, d), jnp.float32),
            pltpu.SemaphoreType.DMA((2,)),
            pltpu.SemaphoreType.DMA((2,)),
            pltpu.SemaphoreType.DMA((2,)),
            pltpu.SemaphoreType.DMA((2,)),
            pltpu.SemaphoreType.DMA((2,)),
            pltpu.SemaphoreType.DMA((2,)),
        ],
    )
    def sc_kernel(y_hbm, init_hbm, src_hbm, dst_hbm, rel_hbm, w_hbm, out_hbm,
                  acc, sv, dv, rv, ia, ib, da, db, wv, ra, rb,
                  ed_sem, w_sem, ga_sem, gb_sem, sa_sem, sb_sem):
        c = lax.axis_index("c")
        s = lax.axis_index("s")
        wid = c * NS + s

        r0 = pl.multiple_of(s * rpt_a, 8)

        @pl.when(c == 0)
        def _():
            @pl.when(s < NS - 1)
            def _():
                pltpu.sync_copy(init_hbm.at[pl.ds(r0, rpt_a)],
                                acc.at[pl.ds(r0, rpt_a)])

            @pl.when(s == NS - 1)
            def _():
                pltpu.sync_copy(init_hbm.at[pl.ds(r0, rpt_b)],
                                acc.at[pl.ds(r0, rpt_b)])

        @pl.when(c == 1)
        def _():
            def zbody(i, _):
                for j in range(seg):
                    ra[0, i, pl.ds(j * LANES, LANES)] = jnp.zeros(
                        (LANES,), jnp.float32)
                return 0

            lax.fori_loop(0, CHUNK, zbody, 0, unroll=False)

            def zfill(row_cnt):
                full = row_cnt // CHUNK
                tail = row_cnt - full * CHUNK
                for t in range(full):
                    off = pl.multiple_of(r0 + t * CHUNK, 8)
                    pltpu.sync_copy(ra.at[0], acc.at[pl.ds(off, CHUNK)])
                if tail:
                    off = pl.multiple_of(r0 + full * CHUNK, 8)
                    pltpu.sync_copy(ra.at[0, pl.ds(0, tail)],
                                    acc.at[pl.ds(off, tail)])

            @pl.when(s < NS - 1)
            def _():
                zfill(rpt_a)

            @pl.when(s == NS - 1)
            def _():
                zfill(rpt_b)

        plsc.subcore_barrier()

        cbase = wid * nc_w

        def fetch_ed(k, slot):
            pltpu.async_copy(src_hbm.at[cbase + k], sv.at[slot],
                             ed_sem.at[slot])
            pltpu.async_copy(dst_hbm.at[cbase + k], dv.at[slot],
                             ed_sem.at[slot])
            pltpu.async_copy(rel_hbm.at[cbase + k], rv.at[slot],
                             ed_sem.at[slot])

        def fetch_w(k, slot):
            pltpu.async_copy(w_hbm.at[cbase + k], wv.at[slot],
                             w_sem.at[slot])

        def wait_ed(k, slot):
            pltpu.make_async_copy(src_hbm.at[cbase + k], sv.at[slot],
                                  ed_sem.at[slot]).wait()
            pltpu.make_async_copy(dst_hbm.at[cbase + k], dv.at[slot],
                                  ed_sem.at[slot]).wait()
            pltpu.make_async_copy(rel_hbm.at[cbase + k], rv.at[slot],
                                  ed_sem.at[slot]).wait()
            pltpu.make_async_copy(w_hbm.at[cbase + k], wv.at[slot],
                                  w_sem.at[slot]).wait()

        def compute_idx(slot):
            for g in range(grp):
                sl = pl.ds(g * LANES, LANES)
                src16 = sv[slot, sl]
                dst16 = dv[slot, sl]
                rbase = rv[slot, sl] * n
                ia[slot, sl] = rbase + src16
                ib[slot, sl] = rbase + dst16
                da[slot, sl] = dst16
                db[slot, sl] = src16

        def issue_gathers(slot):
            pltpu.async_copy(y_hbm.at[ia.at[slot]], ra.at[slot],
                             ga_sem.at[slot])
            pltpu.async_copy(y_hbm.at[ib.at[slot]], rb.at[slot],
                             gb_sem.at[slot])

        def scale(rows, slot):
            def body(g, _):
                w16 = wv[slot, pl.ds(g * LANES, LANES)]
                for l in range(LANES):
                    i = g * LANES + l
                    wb = jnp.full((LANES,), w16[l], jnp.float32)
                    for j in range(seg):
                        sl = pl.ds(j * LANES, LANES)
                        rows[slot, i, sl] = rows[slot, i, sl] * wb
                return 0

            lax.fori_loop(0, grp, body, 0, unroll=False)

        def wait_scatters(slot):
            pltpu.make_async_copy(ra.at[slot], acc.at[da.at[slot]],
                                  sa_sem.at[slot]).wait()
            pltpu.make_async_copy(rb.at[slot], acc.at[db.at[slot]],
                                  sb_sem.at[slot]).wait()

        def step(k, b, prep, wt, prefetch):
            nb = 1 - b
            if prep:
                if wt:
                    wait_scatters(nb)
                wait_ed(k + 1, nb)
                compute_idx(nb)
                if prefetch:
                    fetch_ed(k + 2, b)
                issue_gathers(nb)
            pltpu.make_async_copy(y_hbm.at[ia.at[b]], ra.at[b],
                                  ga_sem.at[b]).wait()
            scale(ra, b)
            pltpu.async_copy(ra.at[b], acc.at[da.at[b]], sa_sem.at[b],
                             add=True)
            pltpu.make_async_copy(y_hbm.at[ib.at[b]], rb.at[b],
                                  gb_sem.at[b]).wait()
            scale(rb, b)
            pltpu.async_copy(rb.at[b], acc.at[db.at[b]], sb_sem.at[b],
                             add=True)
            if prefetch:
                fetch_w(k + 2, b)

        fetch_ed(0, 0)
        fetch_w(0, 0)
        fetch_ed(1, 1)
        fetch_w(1, 1)
        wait_ed(0, 0)
        compute_idx(0)
        issue_gathers(0)
        step(0, 0, prep=True, wt=False, prefetch=True)

        def pair_body(t, _):
            step(2 * t + 1, 1, prep=True, wt=True, prefetch=True)
            step(2 * t + 2, 0, prep=True, wt=True, prefetch=True)
            return 0

        lax.fori_loop(0, (nc_w - 3) // 2, pair_body, 0, unroll=False)

        step(nc_w - 2, 1, prep=True, wt=True, prefetch=False)
        step(nc_w - 1, 0, prep=False, wt=False, prefetch=False)
        wait_scatters(1)
        wait_scatters(0)

        plsc.subcore_barrier()

        @pl.when(s < NS - 1)
        def _():
            pltpu.sync_copy(acc.at[pl.ds(r0, rpt_a)],
                            out_hbm.at[c, pl.ds(r0, rpt_a)])

        @pl.when(s == NS - 1)
        def _():
            pltpu.sync_copy(acc.at[pl.ds(r0, rpt_b)],
                            out_hbm.at[c, pl.ds(r0, rpt_b)])

    return sc_kernel


def kernel(x, node_keep_mask, source, target, edge_type, edge_weights, blocks):
    n, d = x.shape
    r_plus_1, nb, bs, _ = blocks.shape
    r_cnt = r_plus_1 - 1
    e = source.shape[0]

    w_dense = jnp.zeros((r_plus_1, d, d), dtype=jnp.float32)
    for b in range(nb):
        sl = slice(b * bs, (b + 1) * bs)
        w_dense = w_dense.at[:, sl, sl].set(blocks[:, b])

    keepf = node_keep_mask.astype(jnp.float32).reshape(n, 1)

    step = NC * NS * CHUNK
    ep = ((e + step - 1) // step) * step
    if ep // (NC * NS * CHUNK) % 2 == 0:
        ep += step
    if ep != e:
        pad = ep - e
        fill = (jnp.arange(pad, dtype=jnp.int32) * 7) % n
        source = jnp.concatenate([source.astype(jnp.int32), fill])
        target = jnp.concatenate([target.astype(jnp.int32), fill])
        edge_type = jnp.concatenate(
            [edge_type.astype(jnp.int32), jnp.zeros((pad,), jnp.int32)])
        edge_weights = jnp.concatenate(
            [edge_weights, jnp.zeros((pad,), jnp.float32)])
    n_chunks_per_worker = ep // (NC * NS * CHUNK)

    src_hbm = source.astype(jnp.int32).reshape(ep // CHUNK, CHUNK)
    dst_hbm = target.astype(jnp.int32).reshape(ep // CHUNK, CHUNK)
    rel_hbm = edge_type.astype(jnp.int32).reshape(ep // CHUNK, CHUNK)
    w_hbm = edge_weights.reshape(ep // CHUNK, CHUNK)

    tn = 2000
    nt = n // tn

    y, init = pl.pallas_call(
        _transform_kernel,
        grid=(nt,),
        in_specs=[
            pl.BlockSpec((tn, d), lambda i: (i, 0)),
            pl.BlockSpec((r_plus_1, d, d), lambda i: (0, 0, 0)),
            pl.BlockSpec((tn, 1), lambda i: (i, 0)),
        ],
        out_specs=[
            pl.BlockSpec((r_cnt, tn, d), lambda i: (0, i, 0)),
            pl.BlockSpec((tn, d), lambda i: (i, 0)),
        ],
        out_shape=[
            jax.ShapeDtypeStruct((r_cnt, n, d), jnp.float32),
            jax.ShapeDtypeStruct((n, d), jnp.float32),
        ],
    )(x, w_dense, keepf)

    y_flat = y.reshape(r_cnt * n, d)

    sc = _make_sc_kernel(n, d, n_chunks_per_worker)
    partials = sc(y_flat, init, src_hbm, dst_hbm, rel_hbm, w_hbm)

    out = pl.pallas_call(
        _sum_kernel,
        grid=(nt,),
        in_specs=[
            pl.BlockSpec((1, tn, d), lambda i: (0, i, 0)),
            pl.BlockSpec((1, tn, d), lambda i: (1, i, 0)),
        ],
        out_specs=pl.BlockSpec((tn, d), lambda i: (i, 0)),
        out_shape=jax.ShapeDtypeStruct((n, d), jnp.float32),
    )(partials, partials)

    return out
